# baseline (speedup 1.0000x reference)
import functools

import jax
import jax.numpy as jnp
import numpy as np
from jax import lax
from jax.experimental import pallas as pl
from jax.experimental.pallas import tpu as pltpu
from jax.experimental.pallas import tpu_sc as plsc

_RATIOS = [[4, 4], [3, 5], [5, 3], [6, 6], [5, 7], [7, 5], [8, 8], [6, 10],
           [10, 6], [7, 9], [9, 7], [7, 10], [10, 7]]
_STRIDE = 16
_FM = 28
_CAT_NUMS = [2, 3, 2]
_GROUP_RATIOS = [[0, 1, 2], [3, 4, 5], [6, 7, 8, 9, 10, 11, 12]]
_GROUP_LO = [0, 1873, 3458]
_IOU_THRESH = 0.25
_BIG_I = np.int32(1 << 30)
_OFFS = []
_off = 0
for (_kh, _kw) in _RATIOS:
    _OFFS.append(_off)
    _off += (_FM + 1 - _kh) * (_FM + 1 - _kw)


def _tc_body(x_ref, *refs):
    out_refs, mp_ref = refs[:13], refs[13]
    s784 = jnp.sum(x_ref[0], axis=0)
    s = jnp.concatenate(
        [s784[None, _FM * i: _FM * (i + 1)] for i in range(_FM)], axis=0)
    hs = {1: s}
    cur = s
    for k in range(2, 11):
        cur = cur[:, : _FM + 1 - k] + s[:, k - 1:]
        hs[k] = cur
    for r, (kh, kw) in enumerate(_RATIOS):
        h = hs[kw]
        v = h
        for k in range(2, kh + 1):
            v = v[: _FM + 1 - k, :] + h[k - 1:, :]
        v = v / float(kh * kw)
        out_refs[r][0] = v
        mp_ref[0, r] = jnp.pad(v, ((0, kh - 1), (0, kw - 1)),
                               constant_values=-np.inf)


_DNUMS = lax.GatherDimensionNumbers(offset_dims=(),
                                    collapsed_slice_dims=(0,),
                                    start_index_map=(0,))


def _shuffle(v, perm):
    return lax.gather(v, perm[:, None], _DNUMS, (1,),
                      mode=lax.GatherScatterMode.PROMISE_IN_BOUNDS)


def _allmax(v):
    lane = lax.iota(jnp.int32, 16)
    for sh in (1, 2, 4, 8):
        v = jnp.maximum(v, _shuffle(v, jnp.bitwise_xor(lane, sh)))
    return v


def _allmin_i32(v):
    return -_allmax(-v)


def _nms_task(g, b, mp_ref, vm, vi, vs):
    rats = _GROUP_RATIOS[g]
    nr = len(rats)
    cat = _CAT_NUMS[g]
    col0 = sum(_CAT_NUMS[:g])
    r0 = rats[0]
    pltpu.sync_copy(mp_ref.at[pl.ds(b * 13 + r0, nr)], vm.at[0:nr])

    lane = lax.iota(jnp.int32, 16)
    neg = jnp.float32(-np.inf)
    negv = jnp.full((16,), -np.inf, jnp.float32)
    last = jnp.full((16,), _GROUP_LO[g], jnp.int32)
    prev_sel = negv
    acc_i = jnp.zeros((16,), jnp.int32)
    acc_s = jnp.zeros((16,), jnp.float32)

    for t in range(cat):
        ml = negv
        for k in range(nr):
            def _mrow(i, ml, _k=k):
                for q in (0, 12):
                    ml = jnp.maximum(ml, vm[_k, i, pl.ds(q, 16)])
                return ml
            ml = lax.fori_loop(0, _FM, _mrow, ml)
        m = _allmax(ml)
        valid = m != neg
        pl_min = jnp.full((16,), _BIG_I, jnp.int32)
        for k, r in enumerate(rats):
            def _prow(i, carry, _k=k):
                pm, bv = carry
                for q in (0, 12):
                    v = vm[_k, i, pl.ds(q, 16)]
                    pv = bv + (q + lane)
                    pm = jnp.minimum(pm, jnp.where(v == m, pv, _BIG_I))
                return pm, bv + 28
            pl_min, _ = lax.fori_loop(
                0, _FM, _prow,
                (pl_min, jnp.full((16,), r * 784, jnp.int32)))
        ppos = _allmin_i32(pl_min)
        pr = lax.div(ppos, jnp.full((16,), 784, jnp.int32))
        rem = ppos - pr * 784
        pi = lax.div(rem, jnp.full((16,), 28, jnp.int32))
        pj = rem - pi * 28
        kw_s = jnp.zeros((16,), jnp.int32)
        kh_s = jnp.zeros((16,), jnp.int32)
        off_s = jnp.zeros((16,), jnp.int32)
        w2_s = jnp.zeros((16,), jnp.int32)
        for r in rats:
            kh_r, kw_r = _RATIOS[r]
            kw_s = jnp.where(pr == r, kw_r, kw_s)
            kh_s = jnp.where(pr == r, kh_r, kh_s)
            off_s = jnp.where(pr == r, _OFFS[r], off_s)
            w2_s = jnp.where(pr == r, _FM + 1 - kw_r, w2_s)
        abs_idx = off_s + pi * w2_s + pj
        idx = jnp.where(valid, abs_idx, last)
        sel = jnp.where(valid, m, prev_sel)
        acc_i = jnp.where(lane == col0 + t, idx, acc_i)
        acc_s = jnp.where(lane == col0 + t, sel, acc_s)
        if t < cat - 1:
            fx0 = (pj * 16).astype(jnp.float32)
            fy0 = (pi * 16).astype(jnp.float32)
            fx1 = ((pj + kw_s) * 16 - 1).astype(jnp.float32)
            fy1 = ((pi + kh_s) * 16 - 1).astype(jnp.float32)
            far = ((kw_s * 16) * (kh_s * 16)).astype(jnp.float32)
            for k, r in enumerate(rats):
                kh_r, kw_r = _RATIOS[r]
                ar_r = float(kw_r * 16) * float(kh_r * 16)
                def _sup_row(i, carry, _k=k, _kh=kh_r, _kw=kw_r, _ar=ar_r):
                    y0, bv = carry
                    y1 = y0 + jnp.float32(_kh * 16 - 1)
                    ly = jnp.minimum(y1, fy1) - jnp.maximum(y0, fy0) + 1.0
                    for q in (0, 12):
                        xj = lane + q
                        x0 = (xj * 16).astype(jnp.float32)
                        x1 = x0 + jnp.float32(_kw * 16 - 1)
                        lx = jnp.minimum(x1, fx1) - jnp.maximum(x0, fx0) + 1.0
                        inter = jnp.where((lx < 0.0) | (ly < 0.0),
                                          0.0, lx * ly)
                        iou = inter / (_ar + far - inter)
                        pv = bv + (q + lane)
                        kill = (iou > _IOU_THRESH) | (pv == ppos)
                        v = vm[_k, i, pl.ds(q, 16)]
                        vm[_k, i, pl.ds(q, 16)] = jnp.where(
                            jnp.logical_and(valid, kill), negv, v)
                    return y0 + 16.0, bv + 28
                lax.fori_loop(
                    0, _FM, _sup_row,
                    (jnp.zeros((16,), jnp.float32),
                     jnp.full((16,), r * 784, jnp.int32)))
        last = idx
        prev_sel = sel
    vi[0, 0] = acc_i
    vs[0, 0] = acc_s


def _sc_body(mp_ref, idx_ref, sc_ref, vm, vi, vs):
    wid = lax.axis_index("s") * 2 + lax.axis_index("c")
    b = wid // 3
    g = wid % 3

    @pl.when(wid < 12)
    def _work():
        for gs in range(3):
            @pl.when(g == gs)
            def _one(gs=gs):
                _nms_task(gs, b, mp_ref, vm, vi, vs)
                pltpu.sync_copy(vi, idx_ref.at[pl.ds(wid, 1)])
                pltpu.sync_copy(vs, sc_ref.at[pl.ds(wid, 1)])


def _run(x, proposalN):
    b = x.shape[0]
    ch = x.shape[1]
    y = x.reshape(b, ch, _FM * _FM)
    outs = pl.pallas_call(
        _tc_body,
        grid=(b,),
        in_specs=[pl.BlockSpec((1, ch, _FM * _FM), lambda i: (i, 0, 0))],
        out_specs=[pl.BlockSpec((1, _FM + 1 - kh, _FM + 1 - kw),
                                lambda i: (i, 0, 0))
                   for (kh, kw) in _RATIOS]
                  + [pl.BlockSpec((1, 13, _FM, _FM), lambda i: (i, 0, 0, 0))],
        out_shape=[jax.ShapeDtypeStruct((b, _FM + 1 - kh, _FM + 1 - kw),
                                        jnp.float32)
                   for (kh, kw) in _RATIOS]
                  + [jax.ShapeDtypeStruct((b, 13, _FM, _FM), jnp.float32)],
        compiler_params=pltpu.CompilerParams(
            dimension_semantics=("parallel",)),
    )(y)
    pooled, mp = outs[:13], outs[13]
    ws = jnp.concatenate([o.reshape(b, -1) for o in pooled], axis=1)

    nms = functools.partial(
        pl.kernel,
        out_type=[jax.ShapeDtypeStruct((b * 3, 1, 16), jnp.int32),
                  jax.ShapeDtypeStruct((b * 3, 1, 16), jnp.float32)],
        mesh=plsc.VectorSubcoreMesh(core_axis_name="c", subcore_axis_name="s"),
        scratch_types=[pltpu.VMEM((7, _FM, _FM), jnp.float32),
                       pltpu.VMEM((1, 1, 16), jnp.int32),
                       pltpu.VMEM((1, 1, 16), jnp.float32)],
    )(_sc_body)
    idx_o, sc_o = nms(mp.reshape(b * 13, _FM, _FM))
    idx_o = idx_o.reshape(b, 3, 16)
    sc_o = sc_o.reshape(b, 3, 16)

    inds = jnp.concatenate([idx_o[:, g, sum(_CAT_NUMS[:g]):
                                   sum(_CAT_NUMS[:g + 1])] for g in range(3)],
                           axis=1)
    ssc = jnp.concatenate([sc_o[:, g, sum(_CAT_NUMS[:g]):
                                  sum(_CAT_NUMS[:g + 1])] for g in range(3)],
                          axis=1)
    inds = inds + (jnp.asarray(proposalN, jnp.int32) - sum(_CAT_NUMS))
    return inds.astype(jnp.int32), ssc, ws


def kernel(x, proposalN):
    return _run(x, proposalN)


# speedup vs baseline: 1.0028x; 1.0028x over previous
import functools

import jax
import jax.numpy as jnp
import numpy as np
from jax import lax
from jax.experimental import pallas as pl
from jax.experimental.pallas import tpu as pltpu
from jax.experimental.pallas import tpu_sc as plsc

_RATIOS = [[4, 4], [3, 5], [5, 3], [6, 6], [5, 7], [7, 5], [8, 8], [6, 10],
           [10, 6], [7, 9], [9, 7], [7, 10], [10, 7]]
_STRIDE = 16
_FM = 28
_CAT_NUMS = [2, 3, 2]
_GROUP_RATIOS = [[0, 1, 2], [3, 4, 5], [6, 7, 8, 9, 10, 11, 12]]
_GROUP_LO = [0, 1873, 3458]
_IOU_THRESH = 0.25
_BIG_I = np.int32(1 << 30)
_OFFS = []
_off = 0
for (_kh, _kw) in _RATIOS:
    _OFFS.append(_off)
    _off += (_FM + 1 - _kh) * (_FM + 1 - _kw)


def _tc_body(x_ref, *refs):
    out_refs, mp_ref = refs[:13], refs[13]
    s784 = jnp.sum(x_ref[0], axis=0)
    s = jnp.concatenate(
        [s784[None, _FM * i: _FM * (i + 1)] for i in range(_FM)], axis=0)
    hs = {1: s}
    cur = s
    for k in range(2, 11):
        cur = cur[:, : _FM + 1 - k] + s[:, k - 1:]
        hs[k] = cur
    for r, (kh, kw) in enumerate(_RATIOS):
        h = hs[kw]
        v = h
        for k in range(2, kh + 1):
            v = v[: _FM + 1 - k, :] + h[k - 1:, :]
        v = v / float(kh * kw)
        out_refs[r][0] = v
        mp_ref[0, r] = jnp.pad(v, ((0, kh - 1), (0, kw - 1)),
                               constant_values=-np.inf)


_DNUMS = lax.GatherDimensionNumbers(offset_dims=(),
                                    collapsed_slice_dims=(0,),
                                    start_index_map=(0,))


def _shuffle(v, perm):
    return lax.gather(v, perm[:, None], _DNUMS, (1,),
                      mode=lax.GatherScatterMode.PROMISE_IN_BOUNDS)


def _allmax(v):
    lane = lax.iota(jnp.int32, 16)
    for sh in (1, 2, 4, 8):
        v = jnp.maximum(v, _shuffle(v, jnp.bitwise_xor(lane, sh)))
    return v


def _allmin_i32(v):
    return -_allmax(-v)


def _nms_task(g, b, mp_ref, vm, vi, vs):
    rats = _GROUP_RATIOS[g]
    nr = len(rats)
    cat = _CAT_NUMS[g]
    col0 = sum(_CAT_NUMS[:g])
    r0 = rats[0]
    pltpu.sync_copy(mp_ref.at[pl.ds(b * 13 + r0, nr)], vm.at[0:nr])

    lane = lax.iota(jnp.int32, 16)
    neg = jnp.float32(-np.inf)
    negv = jnp.full((16,), -np.inf, jnp.float32)
    last = jnp.full((16,), _GROUP_LO[g], jnp.int32)
    prev_sel = negv
    acc_i = jnp.zeros((16,), jnp.int32)
    acc_s = jnp.zeros((16,), jnp.float32)

    for t in range(cat):
        ml = negv
        pl_min = jnp.full((16,), _BIG_I, jnp.int32)
        for k, r in enumerate(rats):
            h2 = _FM + 1 - _RATIOS[r][0]
            def _mrow(i, carry, _k=k):
                ml_c, pm, bv = carry
                for q in (0, 12):
                    v = vm[_k, i, pl.ds(q, 16)]
                    pv = bv + (q + lane)
                    gt = v > ml_c
                    eq = v == ml_c
                    pm = jnp.where(gt, pv,
                                   jnp.where(eq, jnp.minimum(pm, pv), pm))
                    ml_c = jnp.maximum(ml_c, v)
                return ml_c, pm, bv + 28
            ml, pl_min, _ = lax.fori_loop(
                0, h2, _mrow,
                (ml, pl_min, jnp.full((16,), r * 784, jnp.int32)))
        m = _allmax(ml)
        valid = m != neg
        ppos = _allmin_i32(jnp.where(ml == m, pl_min, _BIG_I))
        pr = lax.div(ppos, jnp.full((16,), 784, jnp.int32))
        rem = ppos - pr * 784
        pi = lax.div(rem, jnp.full((16,), 28, jnp.int32))
        pj = rem - pi * 28
        kw_s = jnp.zeros((16,), jnp.int32)
        kh_s = jnp.zeros((16,), jnp.int32)
        off_s = jnp.zeros((16,), jnp.int32)
        w2_s = jnp.zeros((16,), jnp.int32)
        for r in rats:
            kh_r, kw_r = _RATIOS[r]
            kw_s = jnp.where(pr == r, kw_r, kw_s)
            kh_s = jnp.where(pr == r, kh_r, kh_s)
            off_s = jnp.where(pr == r, _OFFS[r], off_s)
            w2_s = jnp.where(pr == r, _FM + 1 - kw_r, w2_s)
        abs_idx = off_s + pi * w2_s + pj
        idx = jnp.where(valid, abs_idx, last)
        sel = jnp.where(valid, m, prev_sel)
        acc_i = jnp.where(lane == col0 + t, idx, acc_i)
        acc_s = jnp.where(lane == col0 + t, sel, acc_s)
        if t < cat - 1:
            fx0 = (pj * 16).astype(jnp.float32)
            fy0 = (pi * 16).astype(jnp.float32)
            fx1 = ((pj + kw_s) * 16 - 1).astype(jnp.float32)
            fy1 = ((pi + kh_s) * 16 - 1).astype(jnp.float32)
            far = ((kw_s * 16) * (kh_s * 16)).astype(jnp.float32)
            for k, r in enumerate(rats):
                kh_r, kw_r = _RATIOS[r]
                ar_r = float(kw_r * 16) * float(kh_r * 16)
                def _sup_row(i, carry, _k=k, _kh=kh_r, _kw=kw_r, _ar=ar_r):
                    y0, bv = carry
                    y1 = y0 + jnp.float32(_kh * 16 - 1)
                    ly = jnp.minimum(y1, fy1) - jnp.maximum(y0, fy0) + 1.0
                    for q in (0, 12):
                        xj = lane + q
                        x0 = (xj * 16).astype(jnp.float32)
                        x1 = x0 + jnp.float32(_kw * 16 - 1)
                        lx = jnp.minimum(x1, fx1) - jnp.maximum(x0, fx0) + 1.0
                        inter = jnp.where((lx < 0.0) | (ly < 0.0),
                                          0.0, lx * ly)
                        iou = inter / (_ar + far - inter)
                        pv = bv + (q + lane)
                        kill = (iou > _IOU_THRESH) | (pv == ppos)
                        v = vm[_k, i, pl.ds(q, 16)]
                        vm[_k, i, pl.ds(q, 16)] = jnp.where(
                            jnp.logical_and(valid, kill), negv, v)
                    return y0 + 16.0, bv + 28
                lax.fori_loop(
                    0, _FM + 1 - kh_r, _sup_row,
                    (jnp.zeros((16,), jnp.float32),
                     jnp.full((16,), r * 784, jnp.int32)))
        last = idx
        prev_sel = sel
    vi[0, 0] = acc_i
    vs[0, 0] = acc_s


def _sc_body(mp_ref, idx_ref, sc_ref, vm, vi, vs):
    wid = lax.axis_index("s") * 2 + lax.axis_index("c")
    b = wid // 3
    g = wid % 3

    @pl.when(wid < 12)
    def _work():
        for gs in range(3):
            @pl.when(g == gs)
            def _one(gs=gs):
                _nms_task(gs, b, mp_ref, vm, vi, vs)
                pltpu.sync_copy(vi, idx_ref.at[pl.ds(wid, 1)])
                pltpu.sync_copy(vs, sc_ref.at[pl.ds(wid, 1)])


def _run(x, proposalN):
    b = x.shape[0]
    ch = x.shape[1]
    y = x.reshape(b, ch, _FM * _FM)
    outs = pl.pallas_call(
        _tc_body,
        grid=(b,),
        in_specs=[pl.BlockSpec((1, ch, _FM * _FM), lambda i: (i, 0, 0))],
        out_specs=[pl.BlockSpec((1, _FM + 1 - kh, _FM + 1 - kw),
                                lambda i: (i, 0, 0))
                   for (kh, kw) in _RATIOS]
                  + [pl.BlockSpec((1, 13, _FM, _FM), lambda i: (i, 0, 0, 0))],
        out_shape=[jax.ShapeDtypeStruct((b, _FM + 1 - kh, _FM + 1 - kw),
                                        jnp.float32)
                   for (kh, kw) in _RATIOS]
                  + [jax.ShapeDtypeStruct((b, 13, _FM, _FM), jnp.float32)],
        compiler_params=pltpu.CompilerParams(
            dimension_semantics=("parallel",)),
    )(y)
    pooled, mp = outs[:13], outs[13]
    ws = jnp.concatenate([o.reshape(b, -1) for o in pooled], axis=1)

    nms = functools.partial(
        pl.kernel,
        out_type=[jax.ShapeDtypeStruct((b * 3, 1, 16), jnp.int32),
                  jax.ShapeDtypeStruct((b * 3, 1, 16), jnp.float32)],
        mesh=plsc.VectorSubcoreMesh(core_axis_name="c", subcore_axis_name="s"),
        scratch_types=[pltpu.VMEM((7, _FM, _FM), jnp.float32),
                       pltpu.VMEM((1, 1, 16), jnp.int32),
                       pltpu.VMEM((1, 1, 16), jnp.float32)],
    )(_sc_body)
    idx_o, sc_o = nms(mp.reshape(b * 13, _FM, _FM))
    idx_o = idx_o.reshape(b, 3, 16)
    sc_o = sc_o.reshape(b, 3, 16)

    inds = jnp.concatenate([idx_o[:, g, sum(_CAT_NUMS[:g]):
                                   sum(_CAT_NUMS[:g + 1])] for g in range(3)],
                           axis=1)
    ssc = jnp.concatenate([sc_o[:, g, sum(_CAT_NUMS[:g]):
                                  sum(_CAT_NUMS[:g + 1])] for g in range(3)],
                          axis=1)
    inds = inds + (jnp.asarray(proposalN, jnp.int32) - sum(_CAT_NUMS))
    return inds.astype(jnp.int32), ssc, ws


def kernel(x, proposalN):
    return _run(x, proposalN)
